# baseline (speedup 1.0000x reference)
import jax
import jax.numpy as jnp
from jax import lax
from jax.experimental import pallas as pl
from jax.experimental.pallas import tpu as pltpu

B, N, G, M = 8, 16384, 512, 64


def _fps_kernel(xs_ref, ys_ref, zs_ref, cx_ref, cy_ref, cz_ref, dists_ref):
    xs = xs_ref[...]
    ys = ys_ref[...]
    zs = zs_ref[...]
    lx = xs[:, 0:1]
    ly = ys[:, 0:1]
    lz = zs[:, 0:1]
    cx_ref[0:1, :] = lx.reshape(1, B)
    cy_ref[0:1, :] = ly.reshape(1, B)
    cz_ref[0:1, :] = lz.reshape(1, B)
    dists_ref[...] = jnp.full((B, N), 1e10, dtype=jnp.float32)
    iota = lax.broadcasted_iota(jnp.int32, (B, N), 1)

    def body(i, carry):
        lx, ly, lz = carry
        dx = xs - lx
        dy = ys - ly
        dz = zs - lz
        d = dx * dx + dy * dy + dz * dz
        dists = jnp.minimum(dists_ref[...], d)
        dists_ref[...] = dists
        m = jnp.max(dists, axis=1, keepdims=True)
        idx = jnp.min(jnp.where(dists == m, iota, N), axis=1, keepdims=True)
        sel = iota == idx
        nlx = jnp.sum(jnp.where(sel, xs, 0.0), axis=1, keepdims=True)
        nly = jnp.sum(jnp.where(sel, ys, 0.0), axis=1, keepdims=True)
        nlz = jnp.sum(jnp.where(sel, zs, 0.0), axis=1, keepdims=True)
        cx_ref[pl.ds(i, 1), :] = nlx.reshape(1, B)
        cy_ref[pl.ds(i, 1), :] = nly.reshape(1, B)
        cz_ref[pl.ds(i, 1), :] = nlz.reshape(1, B)
        return (nlx, nly, nlz)

    lax.fori_loop(1, G, body, (lx, ly, lz))


def _fps(xyz):
    xs = xyz[:, :, 0]
    ys = xyz[:, :, 1]
    zs = xyz[:, :, 2]
    cx, cy, cz = pl.pallas_call(
        _fps_kernel,
        out_shape=[jax.ShapeDtypeStruct((G, B), jnp.float32)] * 3,
        scratch_shapes=[pltpu.VMEM((B, N), jnp.float32)],
    )(xs, ys, zs)
    return jnp.stack([cx.T, cy.T, cz.T], axis=-1)


def _dist_kernel(c_ref, xt_ref, d_ref):
    c = c_ref[0]
    xt = xt_ref[0]
    s = jax.lax.dot_general(c, xt, (((1,), (0,)), ((), ())),
                            preferred_element_type=jnp.float32)
    dist = -2.0 * s
    cn = jnp.sum(c * c, axis=1, keepdims=True)
    xn = jnp.sum(xt * xt, axis=0, keepdims=True)
    dist = dist + cn
    dist = dist + xn
    d_ref[0] = dist


def _distances(center, xyz):
    xt = jnp.transpose(xyz, (0, 2, 1))
    GB = 128
    d = pl.pallas_call(
        _dist_kernel,
        grid=(B, G // GB),
        in_specs=[
            pl.BlockSpec((1, GB, 3), lambda b, g: (b, g, 0)),
            pl.BlockSpec((1, 3, N), lambda b, g: (b, 0, 0)),
        ],
        out_specs=pl.BlockSpec((1, GB, N), lambda b, g: (b, g, 0)),
        out_shape=jax.ShapeDtypeStruct((B, G, N), jnp.float32),
    )(center, xt)
    return d


def kernel(xyz):
    center = _fps(xyz)
    idx = jnp.broadcast_to(jnp.arange(M, dtype=jnp.int32)[None, None, :], (B, G, M))
    neighborhood = jax.vmap(lambda pts, i: pts[i])(xyz, idx)
    neighborhood = neighborhood - center[:, :, None, :]
    return (neighborhood, center)


# speedup vs baseline: 1.8114x; 1.8114x over previous
import functools

import jax
import jax.numpy as jnp
from jax import lax
from jax.experimental import pallas as pl
from jax.experimental.pallas import tpu as pltpu
from jax.experimental.pallas import tpu_sc as plsc

B, N, G, M = 8, 16384, 512, 64


def _fps_kernel(xs_ref, ys_ref, zs_ref, cx_ref, cy_ref, cz_ref, dists_ref):
    xs = xs_ref[...]
    ys = ys_ref[...]
    zs = zs_ref[...]
    lx = xs[:, 0:1]
    ly = ys[:, 0:1]
    lz = zs[:, 0:1]
    cx_ref[0:1, :] = lx.reshape(1, B)
    cy_ref[0:1, :] = ly.reshape(1, B)
    cz_ref[0:1, :] = lz.reshape(1, B)
    dists_ref[...] = jnp.full((B, N), 1e10, dtype=jnp.float32)
    iota = lax.broadcasted_iota(jnp.int32, (B, N), 1)

    def body(i, carry):
        lx, ly, lz = carry
        dx = xs - lx
        dy = ys - ly
        dz = zs - lz
        d = dx * dx + dy * dy + dz * dz
        dists = jnp.minimum(dists_ref[...], d)
        dists_ref[...] = dists
        m = jnp.max(dists, axis=1, keepdims=True)
        idx = jnp.min(jnp.where(dists == m, iota, N), axis=1, keepdims=True)
        sel = iota == idx
        nlx = jnp.sum(jnp.where(sel, xs, 0.0), axis=1, keepdims=True)
        nly = jnp.sum(jnp.where(sel, ys, 0.0), axis=1, keepdims=True)
        nlz = jnp.sum(jnp.where(sel, zs, 0.0), axis=1, keepdims=True)
        cx_ref[pl.ds(i, 1), :] = nlx.reshape(1, B)
        cy_ref[pl.ds(i, 1), :] = nly.reshape(1, B)
        cz_ref[pl.ds(i, 1), :] = nlz.reshape(1, B)
        return (nlx, nly, nlz)

    lax.fori_loop(1, G, body, (lx, ly, lz))


def _fps(xyz):
    xs = xyz[:, :, 0]
    ys = xyz[:, :, 1]
    zs = xyz[:, :, 2]
    cx, cy, cz = pl.pallas_call(
        _fps_kernel,
        out_shape=[jax.ShapeDtypeStruct((G, B), jnp.float32)] * 3,
        scratch_shapes=[pltpu.VMEM((B, N), jnp.float32)],
    )(xs, ys, zs)
    return jnp.stack([cx.T, cy.T, cz.T], axis=-1)


def _dist_kernel(c_ref, xt_ref, d_ref):
    c = c_ref[0]
    xt = xt_ref[0]
    s = jax.lax.dot_general(c, xt, (((1,), (0,)), ((), ())),
                            preferred_element_type=jnp.float32)
    dist = -2.0 * s
    cn = jnp.sum(c * c, axis=1, keepdims=True)
    xn = jnp.sum(xt * xt, axis=0, keepdims=True)
    dist = dist + cn
    dist = dist + xn
    d_ref[0] = dist


def _distances(center, xyz):
    xt = jnp.transpose(xyz, (0, 2, 1))
    GB = 128
    d = pl.pallas_call(
        _dist_kernel,
        grid=(B, G // GB),
        in_specs=[
            pl.BlockSpec((1, GB, 3), lambda b, g: (b, g, 0)),
            pl.BlockSpec((1, 3, N), lambda b, g: (b, 0, 0)),
        ],
        out_specs=pl.BlockSpec((1, GB, N), lambda b, g: (b, g, 0)),
        out_shape=jax.ShapeDtypeStruct((B, G, N), jnp.float32),
    )(center, xt)
    return d


NC, NS = 2, 16
NW = NC * NS
RPW = (B * G) // NW
CAP = 192
CHUNK = 128
NCHUNK = (N - CHUNK) // CHUNK
INF = float("inf")


def _vs(a):
    k, v = plsc.sort_key_val(a[0], a[1])
    return (k, v)


def _rev(a):
    return (lax.rev(a[0], (0,)), lax.rev(a[1], (0,)))


def _ce(a, b):
    m = b[0] < a[0]
    lo = (jnp.where(m, b[0], a[0]), jnp.where(m, b[1], a[1]))
    hi = (jnp.where(m, a[0], b[0]), jnp.where(m, a[1], b[1]))
    return lo, hi


def _merge2(a, b):
    lo, hi = _ce(a, _rev(b))
    return [_vs(lo), _vs(hi)]


def _sort_bitonic32(x):
    lo, hi = _ce(x[0], x[1])
    return [_vs(lo), _vs(hi)]


def _merge4(a, b):
    rb = [_rev(b[1]), _rev(b[0])]
    l0, h0 = _ce(a[0], rb[0])
    l1, h1 = _ce(a[1], rb[1])
    return _sort_bitonic32([l0, l1]) + _sort_bitonic32([h0, h1])


def _sort64(v):
    s = [_vs(x) for x in v]
    return _merge4(_merge2(s[0], s[1]), _merge2(s[2], s[3]))


def _low64(a, b):
    rb = [_rev(b[3]), _rev(b[2]), _rev(b[1]), _rev(b[0])]
    ls = [_ce(a[k], rb[k])[0] for k in range(4)]
    l0, l2 = _ce(ls[0], ls[2])
    l1, l3 = _ce(ls[1], ls[3])
    a0, a1 = _ce(l0, l1)
    a2, a3 = _ce(l2, l3)
    return [_vs(a0), _vs(a1), _vs(a2), _vs(a3)]


def _pack_cur(cur):
    out = []
    for p in cur:
        out.extend(p)
    return tuple(out)


def _unpack_cur(flat):
    return [(flat[2 * k], flat[2 * k + 1]) for k in range(4)]


def _sc_body(sq_hbm, xs_hbm, ys_hbm, zs_hbm, cx_hbm, cy_hbm, cz_hbm, out_hbm,
             xsb, ysb, zsb, rowbuf, cbx, cby, cbz, pd, pi, outb, scr):
    c = lax.axis_index("c")
    s = lax.axis_index("s")
    wid = s * NC + c
    row0 = wid * RPW
    b = row0 // G
    pltpu.sync_copy(xs_hbm.at[b], xsb)
    pltpu.sync_copy(ys_hbm.at[b], ysb)
    pltpu.sync_copy(zs_hbm.at[b], zsb)
    pltpu.sync_copy(cx_hbm.at[pl.ds(row0, RPW)], cbx)
    pltpu.sync_copy(cy_hbm.at[pl.ds(row0, RPW)], cby)
    pltpu.sync_copy(cz_hbm.at[pl.ds(row0, RPW)], cbz)
    iota16 = lax.iota(jnp.int32, 16)
    inf16 = jnp.full((16,), INF, jnp.float32)
    for k in range(CAP // 16):
        pd[pl.ds(k * 16, 16)] = inf16

    def splat_last(v):
        scr[...] = v
        return plsc.load_gather(scr, [jnp.full((16,), 15, jnp.int32)])

    def pend_low64():
        p = [(pd[pl.ds(k * 16, 16)], pi[pl.ds(k * 16, 16)])
             for k in range(CAP // 16)]
        s1 = _sort64(p[0:4])
        s2 = _sort64(p[4:8])
        s3 = _sort64(p[8:12])
        return _low64(_low64(s1, s2), s3)

    def reset_pend():
        for k in range(CAP // 16):
            pd[pl.ds(k * 16, 16)] = inf16

    def do_row(r, _):
        row = row0 + r
        pltpu.sync_copy(sq_hbm.at[row], rowbuf)
        v = [(rowbuf[pl.ds(k * 16, 16)], iota16 + (k * 16)) for k in range(8)]
        cur = _low64(_sort64(v[0:4]), _sort64(v[4:8]))
        taud = splat_last(cur[3][0])

        def chunk_step(ch, carry):
            pcs = carry[0]
            taud = carry[1]
            cur_flat = carry[2:]
            base = CHUNK + ch * CHUNK
            ccv = jnp.zeros((16,), jnp.int32)
            for k in range(CHUNK // 16):
                dv = rowbuf[pl.ds(base + k * 16, 16)]
                ccv = ccv + plsc.all_reduce_population_count(dv < taud)
            cc = ccv[0]

            def slow(pcs, taud, *flat):
                pv = jnp.full((16,), pcs, jnp.int32)
                for k in range(CHUNK // 16):
                    dv = rowbuf[pl.ds(base + k * 16, 16)]
                    mk = dv < taud
                    cum = plsc.cumsum(jnp.where(mk, 1, 0))
                    pos = pv + cum - 1
                    plsc.store_scatter(pd, [pos], dv, mask=mk)
                    iv = iota16 + (base + k * 16)
                    plsc.store_scatter(pi, [pos], iv, mask=mk)
                    pv = pv + plsc.all_reduce_population_count(mk)
                pcs2 = pcs + cc

                def merge(pcs2, taud, *flat2):
                    cur = _unpack_cur(flat2)
                    ncur = _low64(cur, pend_low64())
                    reset_pend()
                    ntau = splat_last(ncur[3][0])
                    return (0, ntau) + _pack_cur(ncur)

                def nomerge(pcs2, taud, *flat2):
                    return (pcs2, taud) + flat2

                return lax.cond(pcs2 >= 64, merge, nomerge,
                                pcs2, taud, *flat)

            def fast(pcs, taud, *flat):
                return (pcs, taud) + flat

            return lax.cond(cc > 0, slow, fast, pcs, taud, *cur_flat)

        carry = lax.fori_loop(
            0, NCHUNK, chunk_step, (0, taud) + _pack_cur(cur))
        pcs = carry[0]

        def final_merge(*flat):
            cur = _unpack_cur(flat)
            ncur = _low64(cur, pend_low64())
            reset_pend()
            return _pack_cur(ncur)

        def final_keep(*flat):
            return flat

        cur = _unpack_cur(lax.cond(pcs > 0, final_merge, final_keep,
                                   *carry[2:]))

        rl = jnp.full((16,), r, jnp.int32)
        cxs = plsc.load_gather(cbx, [rl])
        cys = plsc.load_gather(cby, [rl])
        czs = plsc.load_gather(cbz, [rl])
        for k in range(4):
            iv = cur[k][1]
            gx = plsc.load_gather(xsb, [iv])
            gy = plsc.load_gather(ysb, [iv])
            gz = plsc.load_gather(zsb, [iv])
            pos = (iota16 + (k * 16)) * 3
            plsc.store_scatter(outb, [pos], gx - cxs)
            plsc.store_scatter(outb, [pos + 1], gy - cys)
            plsc.store_scatter(outb, [pos + 2], gz - czs)
        pltpu.sync_copy(outb, out_hbm.at[row])
        return 0

    lax.fori_loop(0, RPW, do_row, 0)


def _topk_gather(sqrdists, xyz, center):
    mesh = plsc.VectorSubcoreMesh(core_axis_name="c", subcore_axis_name="s",
                                  num_cores=NC, num_subcores=NS)
    f = pl.kernel(
        _sc_body,
        out_type=jax.ShapeDtypeStruct((B * G, M * 3), jnp.float32),
        mesh=mesh,
        compiler_params=pltpu.CompilerParams(needs_layout_passes=False),
        scratch_types=[
            pltpu.VMEM((N,), jnp.float32),
            pltpu.VMEM((N,), jnp.float32),
            pltpu.VMEM((N,), jnp.float32),
            pltpu.VMEM((N,), jnp.float32),
            pltpu.VMEM((RPW,), jnp.float32),
            pltpu.VMEM((RPW,), jnp.float32),
            pltpu.VMEM((RPW,), jnp.float32),
            pltpu.VMEM((CAP,), jnp.float32),
            pltpu.VMEM((CAP,), jnp.int32),
            pltpu.VMEM((M * 3,), jnp.float32),
            pltpu.VMEM((16,), jnp.float32),
        ],
    )
    cf = center.reshape(B * G, 3)
    out = f(sqrdists.reshape(B * G, N),
            xyz[:, :, 0], xyz[:, :, 1], xyz[:, :, 2],
            cf[:, 0], cf[:, 1], cf[:, 2])
    return out.reshape(B, G, M, 3)


def kernel(xyz):
    center = _fps(xyz)
    sqrdists = _distances(center, xyz)
    neighborhood = _topk_gather(sqrdists, xyz, center)
    return (neighborhood, center)
